# baseline (speedup 1.0000x reference)
import functools

import jax
import jax.numpy as jnp
from jax import lax
from jax.experimental import pallas as pl
from jax.experimental.pallas import tpu as pltpu
from jax.experimental.pallas import tpu_sc as plsc

NUM_USERS = 25000
NUM_ITEMS = 25000
EMBED_DIM = 64
FEAT = 32
N_NODES = NUM_USERS + NUM_ITEMS
N_EDGES = 800000

NUM_CORES = 2
NUM_SUBCORES = 16
LANES = 16

GARBAGE_ROW = N_NODES
N_TAB_ROWS = 50176
ROWS_PER_TILE = N_TAB_ROWS // NUM_SUBCORES

CHUNK = 128
SUP = 4
CHUNKS = 392
NSUP = CHUNKS // SUP
G = 5
LSLOT = 4
EDGES_PER_TILE = CHUNKS * CHUNK
E_PAD = NUM_SUBCORES * EDGES_PER_TILE
EDGE_ROWS = E_PAD // CHUNK
ROWS_T = EDGES_PER_TILE // CHUNK


def _layer_body(col0_ref, col1_ref, idx_ref, val_ref, tab_ref, out_ref,
                colb, idxb, valb, gbig, acc,
                sem_ld, sem_g, sem_sc):
    c = lax.axis_index("c")
    s = lax.axis_index("s")
    tile_row0 = s * ROWS_PER_TILE
    erow0 = s * ROWS_T

    zero16 = jnp.zeros((LANES,), jnp.float32)

    def _zfill(k, carry):
        r = k // 2
        d = lax.rem(k, 2)
        gbig[r, pl.ds(d * LANES, LANES)] = zero16
        return carry

    lax.fori_loop(0, CHUNK * 2, _zfill, 0)
    zcopies = []
    for q in range(ROWS_PER_TILE // CHUNK):
        zcopies.append(pltpu.async_copy(
            gbig.at[pl.ds(0, CHUNK)],
            acc.at[pl.ds(tile_row0 + q * CHUNK, CHUNK)], sem_sc))
    zrem = ROWS_PER_TILE % CHUNK
    if zrem:
        zcopies.append(pltpu.async_copy(
            gbig.at[pl.ds(0, zrem)],
            acc.at[pl.ds(tile_row0 + (ROWS_PER_TILE // CHUNK) * CHUNK, zrem)],
            sem_sc))
    for d_ in zcopies:
        d_.wait()
    plsc.subcore_barrier()

    def _issue_loads(sup, slot):
        roff = erow0 + sup * SUP

        @pl.when(c == 0)
        def _():
            pltpu.async_copy(col0_ref.at[pl.ds(roff, SUP)], colb.at[slot], sem_ld)

        @pl.when(c == 1)
        def _():
            pltpu.async_copy(col1_ref.at[pl.ds(roff, SUP)], colb.at[slot], sem_ld)

        pltpu.async_copy(idx_ref.at[pl.ds(roff, SUP)], idxb.at[slot], sem_ld)
        pltpu.async_copy(val_ref.at[pl.ds(roff, SUP)], valb.at[slot], sem_ld)

    def _wait_loads(slot):
        pltpu.make_async_copy(
            col0_ref.at[pl.ds(0, SUP)], colb.at[slot], sem_ld).wait()
        pltpu.make_async_copy(
            idx_ref.at[pl.ds(0, SUP)], idxb.at[slot], sem_ld).wait()
        pltpu.make_async_copy(
            val_ref.at[pl.ds(0, SUP)], valb.at[slot], sem_ld).wait()

    def _gather_desc(k):
        sup = k // SUP
        return pltpu.make_async_copy(
            tab_ref.at[colb.at[sup % LSLOT, k % SUP]],
            gbig.at[pl.ds((k % G) * CHUNK, CHUNK)], sem_g)

    def _scatter_desc(k):
        sup = k // SUP
        return pltpu.make_async_copy(
            gbig.at[pl.ds((k % G) * CHUNK, CHUNK)],
            acc.at[idxb.at[sup % LSLOT, k % SUP]], sem_sc)

    for sup0 in range(LSLOT - 1):
        _issue_loads(sup0, sup0)
    _wait_loads(0)
    for k0 in range(3):
        _gather_desc(k0).start()

    def _chunk(k, carry):
        sup = k // SUP
        b = k % SUP
        lsl = sup % LSLOT
        base = (k % G) * CHUNK
        _gather_desc(k).wait()

        def _grp(g, carry2):
            vals = valb[lsl, b, pl.ds(g * LANES, LANES)]
            for e in range(LANES):
                row = base + g * LANES + e
                vb = jnp.full((LANES,), vals[e], jnp.float32)
                for d in range(FEAT // LANES):
                    sl = pl.ds(d * LANES, LANES)
                    gbig[row, sl] = gbig[row, sl] * vb
            return carry2

        lax.fori_loop(0, CHUNK // LANES, _grp, 0)

        _scatter_desc(k).start(add=True)

        @pl.when(k >= 2)
        def _():
            _scatter_desc(k - 2).wait()

        @pl.when(k + 3 < CHUNKS)
        def _():
            @pl.when((k + 3) % SUP == 0)
            def _():
                _wait_loads(((k + 3) // SUP) % LSLOT)

            _gather_desc(k + 3).start()

        @pl.when((b == 2) & (sup + 3 < NSUP))
        def _():
            _issue_loads(sup + 3, (sup + 3) % LSLOT)

        return carry

    lax.fori_loop(0, CHUNKS, _chunk, 0)
    for ktail in (CHUNKS - 2, CHUNKS - 1):
        _scatter_desc(ktail).wait()
    plsc.subcore_barrier()

    out_row0 = c * N_TAB_ROWS + tile_row0
    pltpu.sync_copy(acc.at[pl.ds(tile_row0, ROWS_PER_TILE)],
                    out_ref.at[pl.ds(out_row0, ROWS_PER_TILE)])


@functools.partial(
    pl.kernel,
    out_type=jax.ShapeDtypeStruct((NUM_CORES * N_TAB_ROWS, FEAT), jnp.float32),
    mesh=plsc.VectorSubcoreMesh(core_axis_name="c", subcore_axis_name="s"),
    compiler_params=pltpu.CompilerParams(use_tc_tiling_on_sc=False),
    scratch_types=[
        pltpu.VMEM((LSLOT, SUP, CHUNK), jnp.int32),
        pltpu.VMEM((LSLOT, SUP, CHUNK), jnp.int32),
        pltpu.VMEM((LSLOT, SUP, CHUNK), jnp.float32),
        pltpu.VMEM((G * CHUNK, FEAT), jnp.float32),
        pltpu.VMEM_SHARED((N_TAB_ROWS, FEAT), jnp.float32),
        pltpu.SemaphoreType.DMA,
        pltpu.SemaphoreType.DMA,
        pltpu.SemaphoreType.DMA,
    ],
)
def _layer(col0_ref, col1_ref, idx_ref, val_ref, tab_ref, out_ref,
           colb, idxb, valb, gbig, acc,
           sem_ld, sem_g, sem_sc):
    _layer_body(col0_ref, col1_ref, idx_ref, val_ref, tab_ref, out_ref,
                colb, idxb, valb, gbig, acc,
                sem_ld, sem_g, sem_sc)


def _mean_body(a_ref, b_ref, c_ref, d_ref, o_ref):
    o_ref[...] = (a_ref[...] + b_ref[...] + c_ref[...] + d_ref[...]) * 0.25


def _mean4(t0, t1, t2, t3):
    spec = pl.BlockSpec((ROWS_PER_TILE, FEAT), lambda i: (i, 0))
    return pl.pallas_call(
        _mean_body,
        grid=(NUM_CORES * N_TAB_ROWS // ROWS_PER_TILE,),
        in_specs=[spec, spec, spec, spec],
        out_specs=spec,
        out_shape=jax.ShapeDtypeStruct((NUM_CORES * N_TAB_ROWS, FEAT),
                                       jnp.float32),
    )(t0, t1, t2, t3)


def kernel(adj_indices, adj_values, user_embeds, item_embeds):
    row = adj_indices[0]
    col = adj_indices[1]
    pad_n = E_PAD - N_EDGES
    col0 = jnp.concatenate([col, jnp.zeros((pad_n,), jnp.int32)])
    col1 = col0 + N_TAB_ROWS
    idxr = jnp.concatenate([row, jnp.full((pad_n,), GARBAGE_ROW, jnp.int32)])
    valp = jnp.concatenate([adj_values, jnp.zeros((pad_n,), jnp.float32)])
    col0 = col0.reshape(EDGE_ROWS, CHUNK)
    col1 = col1.reshape(EDGE_ROWS, CHUNK)
    idxr = idxr.reshape(EDGE_ROWS, CHUNK)
    valp = valp.reshape(EDGE_ROWS, CHUNK)

    pad_blk = jnp.zeros((N_TAB_ROWS - N_NODES, FEAT), jnp.float32)
    t0 = jnp.concatenate([
        user_embeds[:, :FEAT], item_embeds[:, :FEAT], pad_blk,
        user_embeds[:, FEAT:], item_embeds[:, FEAT:], pad_blk,
    ], axis=0)

    t1 = _layer(col0, col1, idxr, valp, t0)
    t2 = _layer(col0, col1, idxr, valp, t1)
    t3 = _layer(col0, col1, idxr, valp, t2)

    m = _mean4(t0, t1, t2, t3)
    ml, mr = m[:N_TAB_ROWS], m[N_TAB_ROWS:]
    user_embeddings = jnp.concatenate(
        [ml[:NUM_USERS], mr[:NUM_USERS]], axis=1)
    item_embeddings = jnp.concatenate(
        [ml[NUM_USERS:N_NODES], mr[NUM_USERS:N_NODES]], axis=1)
    return (user_embeddings, item_embeddings)


# speedup vs baseline: 1.5005x; 1.5005x over previous
import functools

import jax
import jax.numpy as jnp
from jax import lax
from jax.experimental import pallas as pl
from jax.experimental.pallas import tpu as pltpu
from jax.experimental.pallas import tpu_sc as plsc

NUM_USERS = 25000
NUM_ITEMS = 25000
EMBED_DIM = 64
FEAT = 32
N_NODES = NUM_USERS + NUM_ITEMS
N_EDGES = 800000

NUM_CORES = 2
NUM_SUBCORES = 16
LANES = 16

GARBAGE_ROW = N_NODES
N_TAB_ROWS = 50176
ROWS_PER_TILE = N_TAB_ROWS // NUM_SUBCORES

CHUNK = 128
SUP = 4
CHUNKS = 392
NSUP = CHUNKS // SUP
EDGES_PER_TILE = CHUNKS * CHUNK
E_PAD = NUM_SUBCORES * EDGES_PER_TILE
EDGE_ROWS = E_PAD // CHUNK
ROWS_T = EDGES_PER_TILE // CHUNK


def _layer_body(col0_ref, col1_ref, idx_ref, val_ref, tab_ref, out_ref,
                colb, idxb, valb, g0, g1, g2, acc,
                sem_ld, sem_g, sem_sc):
    c = lax.axis_index("c")
    s = lax.axis_index("s")
    tile_row0 = s * ROWS_PER_TILE
    erow0 = s * ROWS_T
    gbufs = [g0, g1, g2]

    zero16 = jnp.zeros((LANES,), jnp.float32)

    def _zfill(k, carry):
        r = k // 2
        d = lax.rem(k, 2)
        g0[r, pl.ds(d * LANES, LANES)] = zero16
        return carry

    lax.fori_loop(0, CHUNK * 2, _zfill, 0)
    zcopies = []
    for q in range(ROWS_PER_TILE // CHUNK):
        zcopies.append(pltpu.async_copy(
            g0, acc.at[pl.ds(tile_row0 + q * CHUNK, CHUNK)], sem_sc))
    zrem = ROWS_PER_TILE % CHUNK
    if zrem:
        zcopies.append(pltpu.async_copy(
            g0.at[pl.ds(0, zrem)],
            acc.at[pl.ds(tile_row0 + (ROWS_PER_TILE // CHUNK) * CHUNK, zrem)],
            sem_sc))
    for d_ in zcopies:
        d_.wait()
    plsc.subcore_barrier()

    def _issue_loads(sup, hb):
        roff = erow0 + sup * SUP

        @pl.when(c == 0)
        def _():
            pltpu.async_copy(col0_ref.at[pl.ds(roff, SUP)], colb.at[hb], sem_ld)

        @pl.when(c == 1)
        def _():
            pltpu.async_copy(col1_ref.at[pl.ds(roff, SUP)], colb.at[hb], sem_ld)

        pltpu.async_copy(idx_ref.at[pl.ds(roff, SUP)], idxb.at[hb], sem_ld)
        pltpu.async_copy(val_ref.at[pl.ds(roff, SUP)], valb.at[hb], sem_ld)

    def _wait_loads(hb):
        pltpu.make_async_copy(
            col0_ref.at[pl.ds(0, SUP)], colb.at[hb], sem_ld).wait()
        pltpu.make_async_copy(
            idx_ref.at[pl.ds(0, SUP)], idxb.at[hb], sem_ld).wait()
        pltpu.make_async_copy(
            val_ref.at[pl.ds(0, SUP)], valb.at[hb], sem_ld).wait()

    def _scale(gb, hb, b):
        def _grp(g, carry):
            vals = valb[hb, b, pl.ds(g * LANES, LANES)]
            for e in range(LANES):
                j = g * LANES + e
                vb = jnp.full((LANES,), vals[e], jnp.float32)
                for d in range(FEAT // LANES):
                    sl = pl.ds(d * LANES, LANES)
                    gb[j, sl] = gb[j, sl] * vb
            return carry

        lax.fori_loop(0, CHUNK // LANES, _grp, 0)

    _issue_loads(0, 0)

    def _pair(gg, carry):
        for h in range(2):
            sup = 2 * gg + h
            hb = h
            @pl.when(sup < NSUP - 1)
            def _():
                _issue_loads(sup + 1, 1 - hb)

            _wait_loads(hb)
            gathers = {}
            for b in range(3):
                gathers[b] = pltpu.async_copy(
                    tab_ref.at[colb.at[hb, b]], gbufs[b], sem_g)
            scatters = {}
            for b in range(SUP):
                gathers[b].wait()
                _scale(gbufs[b % 3], hb, b)
                scatters[b] = pltpu.async_copy(
                    gbufs[b % 3], acc.at[idxb.at[hb, b]], sem_sc, add=True)
                if b == 1:
                    scatters[0].wait()
                    gathers[3] = pltpu.async_copy(
                        tab_ref.at[colb.at[hb, 3]], gbufs[0], sem_g)
            for b in range(1, SUP):
                scatters[b].wait()
        return carry

    lax.fori_loop(0, NSUP // 2, _pair, 0)
    plsc.subcore_barrier()

    out_row0 = c * N_TAB_ROWS + tile_row0
    pltpu.sync_copy(acc.at[pl.ds(tile_row0, ROWS_PER_TILE)],
                    out_ref.at[pl.ds(out_row0, ROWS_PER_TILE)])


@functools.partial(
    pl.kernel,
    out_type=jax.ShapeDtypeStruct((NUM_CORES * N_TAB_ROWS, FEAT), jnp.float32),
    mesh=plsc.VectorSubcoreMesh(core_axis_name="c", subcore_axis_name="s"),
    compiler_params=pltpu.CompilerParams(use_tc_tiling_on_sc=False),
    scratch_types=[
        pltpu.VMEM((2, SUP, CHUNK), jnp.int32),
        pltpu.VMEM((2, SUP, CHUNK), jnp.int32),
        pltpu.VMEM((2, SUP, CHUNK), jnp.float32),
        pltpu.VMEM((CHUNK, FEAT), jnp.float32),
        pltpu.VMEM((CHUNK, FEAT), jnp.float32),
        pltpu.VMEM((CHUNK, FEAT), jnp.float32),
        pltpu.VMEM_SHARED((N_TAB_ROWS, FEAT), jnp.float32),
        pltpu.SemaphoreType.DMA,
        pltpu.SemaphoreType.DMA,
        pltpu.SemaphoreType.DMA,
    ],
)
def _layer(col0_ref, col1_ref, idx_ref, val_ref, tab_ref, out_ref,
           colb, idxb, valb, g0, g1, g2, acc,
           sem_ld, sem_g, sem_sc):
    _layer_body(col0_ref, col1_ref, idx_ref, val_ref, tab_ref, out_ref,
                colb, idxb, valb, g0, g1, g2, acc,
                sem_ld, sem_g, sem_sc)


def _mean_body(a_ref, b_ref, c_ref, d_ref, o_ref):
    o_ref[...] = (a_ref[...] + b_ref[...] + c_ref[...] + d_ref[...]) * 0.25


def _mean4(t0, t1, t2, t3):
    spec = pl.BlockSpec((ROWS_PER_TILE, FEAT), lambda i: (i, 0))
    return pl.pallas_call(
        _mean_body,
        grid=(NUM_CORES * N_TAB_ROWS // ROWS_PER_TILE,),
        in_specs=[spec, spec, spec, spec],
        out_specs=spec,
        out_shape=jax.ShapeDtypeStruct((NUM_CORES * N_TAB_ROWS, FEAT),
                                       jnp.float32),
    )(t0, t1, t2, t3)


def kernel(adj_indices, adj_values, user_embeds, item_embeds):
    row = adj_indices[0]
    col = adj_indices[1]
    pad_n = E_PAD - N_EDGES
    col0 = jnp.concatenate([col, jnp.zeros((pad_n,), jnp.int32)])
    col1 = col0 + N_TAB_ROWS
    idxr = jnp.concatenate([row, jnp.full((pad_n,), GARBAGE_ROW, jnp.int32)])
    valp = jnp.concatenate([adj_values, jnp.zeros((pad_n,), jnp.float32)])
    col0 = col0.reshape(EDGE_ROWS, CHUNK)
    col1 = col1.reshape(EDGE_ROWS, CHUNK)
    idxr = idxr.reshape(EDGE_ROWS, CHUNK)
    valp = valp.reshape(EDGE_ROWS, CHUNK)

    pad_blk = jnp.zeros((N_TAB_ROWS - N_NODES, FEAT), jnp.float32)
    t0 = jnp.concatenate([
        user_embeds[:, :FEAT], item_embeds[:, :FEAT], pad_blk,
        user_embeds[:, FEAT:], item_embeds[:, FEAT:], pad_blk,
    ], axis=0)

    t1 = _layer(col0, col1, idxr, valp, t0)
    t2 = _layer(col0, col1, idxr, valp, t1)
    t3 = _layer(col0, col1, idxr, valp, t2)

    m = _mean4(t0, t1, t2, t3)
    ml, mr = m[:N_TAB_ROWS], m[N_TAB_ROWS:]
    user_embeddings = jnp.concatenate(
        [ml[:NUM_USERS], mr[:NUM_USERS]], axis=1)
    item_embeddings = jnp.concatenate(
        [ml[NUM_USERS:N_NODES], mr[NUM_USERS:N_NODES]], axis=1)
    return (user_embeddings, item_embeddings)


# speedup vs baseline: 1.6863x; 1.1238x over previous
import functools

import jax
import jax.numpy as jnp
from jax import lax
from jax.experimental import pallas as pl
from jax.experimental.pallas import tpu as pltpu
from jax.experimental.pallas import tpu_sc as plsc

NUM_USERS = 25000
NUM_ITEMS = 25000
EMBED_DIM = 64
FEAT = 32
N_NODES = NUM_USERS + NUM_ITEMS
N_EDGES = 800000

NUM_CORES = 2
NUM_SUBCORES = 16
LANES = 16

GARBAGE_ROW = N_NODES
N_TAB_ROWS = 50176
ROWS_PER_TILE = N_TAB_ROWS // NUM_SUBCORES

CHUNK = 128
SUP = 4
CHUNKS = 392
NSUP = CHUNKS // SUP
EDGES_PER_TILE = CHUNKS * CHUNK
E_PAD = NUM_SUBCORES * EDGES_PER_TILE
EDGE_ROWS = E_PAD // CHUNK
ROWS_T = EDGES_PER_TILE // CHUNK


def _layer_body(col0_ref, col1_ref, idx_ref, val_ref, tab_ref, out_ref,
                colb, idxb, valb, g0, g1, g2, g3, acc,
                sem_ld, sem_g, sem_sc):
    c = lax.axis_index("c")
    s = lax.axis_index("s")
    tile_row0 = s * ROWS_PER_TILE
    erow0 = s * ROWS_T
    gbufs = [g0, g1, g2, g3]

    zero16 = jnp.zeros((LANES,), jnp.float32)

    def _zfill(k, carry):
        r = k // 2
        d = lax.rem(k, 2)
        g0[r, pl.ds(d * LANES, LANES)] = zero16
        return carry

    lax.fori_loop(0, CHUNK * 2, _zfill, 0)
    zcopies = []
    for q in range(ROWS_PER_TILE // CHUNK):
        zcopies.append(pltpu.async_copy(
            g0, acc.at[pl.ds(tile_row0 + q * CHUNK, CHUNK)], sem_sc))
    zrem = ROWS_PER_TILE % CHUNK
    if zrem:
        zcopies.append(pltpu.async_copy(
            g0.at[pl.ds(0, zrem)],
            acc.at[pl.ds(tile_row0 + (ROWS_PER_TILE // CHUNK) * CHUNK, zrem)],
            sem_sc))
    for d_ in zcopies:
        d_.wait()
    plsc.subcore_barrier()

    def _issue_loads(sup, hb):
        roff = erow0 + sup * SUP

        @pl.when(c == 0)
        def _():
            pltpu.async_copy(col0_ref.at[pl.ds(roff, SUP)], colb.at[hb], sem_ld)

        @pl.when(c == 1)
        def _():
            pltpu.async_copy(col1_ref.at[pl.ds(roff, SUP)], colb.at[hb], sem_ld)

        pltpu.async_copy(idx_ref.at[pl.ds(roff, SUP)], idxb.at[hb], sem_ld)
        pltpu.async_copy(val_ref.at[pl.ds(roff, SUP)], valb.at[hb], sem_ld)

    def _wait_loads(hb):
        pltpu.make_async_copy(
            col0_ref.at[pl.ds(0, SUP)], colb.at[hb], sem_ld).wait()
        pltpu.make_async_copy(
            idx_ref.at[pl.ds(0, SUP)], idxb.at[hb], sem_ld).wait()
        pltpu.make_async_copy(
            val_ref.at[pl.ds(0, SUP)], valb.at[hb], sem_ld).wait()

    def _scale(gb, hb, b):
        def _grp(g, carry):
            vals = valb[hb, b, pl.ds(g * LANES, LANES)]
            for e in range(LANES):
                j = g * LANES + e
                vb = jnp.full((LANES,), vals[e], jnp.float32)
                for d in range(FEAT // LANES):
                    sl = pl.ds(d * LANES, LANES)
                    gb[j, sl] = gb[j, sl] * vb
            return carry

        lax.fori_loop(0, CHUNK // LANES, _grp, 0)

    _issue_loads(0, 0)

    def _pair(gg, carry):
        for h in range(2):
            sup = 2 * gg + h
            hb = h
            @pl.when(sup < NSUP - 1)
            def _():
                _issue_loads(sup + 1, 1 - hb)

            _wait_loads(hb)
            gathers = {}
            for b in range(SUP):
                gathers[b] = pltpu.async_copy(
                    tab_ref.at[colb.at[hb, b]], gbufs[b], sem_g)
            scatters = {}
            for b in range(SUP):
                gathers[b].wait()
                _scale(gbufs[b], hb, b)
                scatters[b] = pltpu.async_copy(
                    gbufs[b], acc.at[idxb.at[hb, b]], sem_sc, add=True)
            for b in range(SUP):
                scatters[b].wait()
        return carry

    lax.fori_loop(0, NSUP // 2, _pair, 0)
    plsc.subcore_barrier()

    out_row0 = c * N_TAB_ROWS + tile_row0
    pltpu.sync_copy(acc.at[pl.ds(tile_row0, ROWS_PER_TILE)],
                    out_ref.at[pl.ds(out_row0, ROWS_PER_TILE)])


@functools.partial(
    pl.kernel,
    out_type=jax.ShapeDtypeStruct((NUM_CORES * N_TAB_ROWS, FEAT), jnp.float32),
    mesh=plsc.VectorSubcoreMesh(core_axis_name="c", subcore_axis_name="s"),
    compiler_params=pltpu.CompilerParams(use_tc_tiling_on_sc=False),
    scratch_types=[
        pltpu.VMEM((2, SUP, CHUNK), jnp.int32),
        pltpu.VMEM((2, SUP, CHUNK), jnp.int32),
        pltpu.VMEM((2, SUP, CHUNK), jnp.float32),
        pltpu.VMEM((CHUNK, FEAT), jnp.float32),
        pltpu.VMEM((CHUNK, FEAT), jnp.float32),
        pltpu.VMEM((CHUNK, FEAT), jnp.float32),
        pltpu.VMEM((CHUNK, FEAT), jnp.float32),
        pltpu.VMEM_SHARED((N_TAB_ROWS, FEAT), jnp.float32),
        pltpu.SemaphoreType.DMA,
        pltpu.SemaphoreType.DMA,
        pltpu.SemaphoreType.DMA,
    ],
)
def _layer(col0_ref, col1_ref, idx_ref, val_ref, tab_ref, out_ref,
           colb, idxb, valb, g0, g1, g2, g3, acc,
           sem_ld, sem_g, sem_sc):
    _layer_body(col0_ref, col1_ref, idx_ref, val_ref, tab_ref, out_ref,
                colb, idxb, valb, g0, g1, g2, g3, acc,
                sem_ld, sem_g, sem_sc)


def _mean_body(a_ref, b_ref, c_ref, d_ref, o_ref):
    o_ref[...] = (a_ref[...] + b_ref[...] + c_ref[...] + d_ref[...]) * 0.25


def _mean4(t0, t1, t2, t3):
    spec = pl.BlockSpec((ROWS_PER_TILE, FEAT), lambda i: (i, 0))
    return pl.pallas_call(
        _mean_body,
        grid=(NUM_CORES * N_TAB_ROWS // ROWS_PER_TILE,),
        in_specs=[spec, spec, spec, spec],
        out_specs=spec,
        out_shape=jax.ShapeDtypeStruct((NUM_CORES * N_TAB_ROWS, FEAT),
                                       jnp.float32),
    )(t0, t1, t2, t3)


def kernel(adj_indices, adj_values, user_embeds, item_embeds):
    row = adj_indices[0]
    col = adj_indices[1]
    pad_n = E_PAD - N_EDGES
    col0 = jnp.concatenate([col, jnp.zeros((pad_n,), jnp.int32)])
    col1 = col0 + N_TAB_ROWS
    idxr = jnp.concatenate([row, jnp.full((pad_n,), GARBAGE_ROW, jnp.int32)])
    valp = jnp.concatenate([adj_values, jnp.zeros((pad_n,), jnp.float32)])
    col0 = col0.reshape(EDGE_ROWS, CHUNK)
    col1 = col1.reshape(EDGE_ROWS, CHUNK)
    idxr = idxr.reshape(EDGE_ROWS, CHUNK)
    valp = valp.reshape(EDGE_ROWS, CHUNK)

    pad_blk = jnp.zeros((N_TAB_ROWS - N_NODES, FEAT), jnp.float32)
    t0 = jnp.concatenate([
        user_embeds[:, :FEAT], item_embeds[:, :FEAT], pad_blk,
        user_embeds[:, FEAT:], item_embeds[:, FEAT:], pad_blk,
    ], axis=0)

    t1 = _layer(col0, col1, idxr, valp, t0)
    t2 = _layer(col0, col1, idxr, valp, t1)
    t3 = _layer(col0, col1, idxr, valp, t2)

    m = _mean4(t0, t1, t2, t3)
    ml, mr = m[:N_TAB_ROWS], m[N_TAB_ROWS:]
    user_embeddings = jnp.concatenate(
        [ml[:NUM_USERS], mr[:NUM_USERS]], axis=1)
    item_embeddings = jnp.concatenate(
        [ml[NUM_USERS:N_NODES], mr[NUM_USERS:N_NODES]], axis=1)
    return (user_embeddings, item_embeddings)
